# baseline (speedup 1.0000x reference)
import functools

import jax
import jax.numpy as jnp
from jax import lax
from jax.experimental import pallas as pl
from jax.experimental.pallas import tpu as pltpu
from jax.experimental.pallas import tpu_sc as plsc

N = 10000
D = 128
HD = 64
E = 320000
NC = 2
NS = 16
NW = NC * NS
CH = 128
NCH = 80
EPW = NCH * CH
EPAD = NW * EPW
RPS = N // NS
DEGW = 16
NACC = N + 8

_MESH = plsc.VectorSubcoreMesh(core_axis_name="c", subcore_axis_name="s",
                               num_cores=NC, num_subcores=NS)
_SC_PARAMS = pltpu.CompilerParams(use_tc_tiling_on_sc=False)


def _zero_rows(ref, nrows, width):
    z = jnp.zeros((16,), jnp.float32)

    def row(i, _):
        for j in range(width // 16):
            ref[i, pl.ds(j * 16, 16)] = z
        return 0

    lax.fori_loop(0, nrows, row, 0)


def _idx_axpy(ref, mul, add):
    def row(i, _):
        for j in range(CH // 16):
            sl = pl.ds(j * 16, 16)
            ref[i, sl] = ref[i, sl] * mul + add
        return 0

    lax.fori_loop(0, NCH, row, 0)


NBUF = 4


def _seg_round(tab, src_v, dst_v, rows, gsems, ssems, acc_sh, deg_tup):
    def g_start(idx, b):
        pltpu.async_copy(tab.at[src_v.at[idx]], rows[b], gsems[b])

    def g_wait(idx, b):
        pltpu.make_async_copy(tab.at[src_v.at[idx]], rows[b], gsems[b]).wait()

    def s_wait(b):
        pltpu.make_async_copy(
            rows[b], acc_sh.at[dst_v.at[0]], ssems[b]).wait()

    def block(idx, b, wait_s, issue_g):
        g_wait(idx, b)
        pltpu.async_copy(rows[b], acc_sh.at[dst_v.at[idx]], ssems[b],
                         add=True)
        if deg_tup is not None:
            ones_v, deg_sh, dsem = deg_tup
            pltpu.async_copy(ones_v, deg_sh.at[dst_v.at[idx]], dsem,
                             add=True)
            if wait_s:
                pltpu.make_async_copy(
                    ones_v, deg_sh.at[dst_v.at[0]], dsem).wait()
        if wait_s:
            s_wait((b - 2) % NBUF)
        if issue_g:
            g_start(idx + 2, (b + 2) % NBUF)

    g_start(0, 0)
    g_start(1, 1)
    block(0, 0, False, True)
    block(1, 1, False, True)

    def quad(j, _):
        i = 4 * j + 2
        for b in (2, 3, 0, 1):
            block(i, b, True, True)
            i = i + 1
        return 0

    nquads = (NCH - 4) // 4
    lax.fori_loop(0, nquads, quad, 0)
    for i in range(2 + 4 * nquads, NCH):
        block(i, i % NBUF, True, i + 2 <= NCH - 1)
    s_wait((NCH - 2) % NBUF)
    s_wait((NCH - 1) % NBUF)
    if deg_tup is not None:
        ones_v, deg_sh, dsem = deg_tup
        for _ in range(2):
            pltpu.make_async_copy(
                ones_v, deg_sh.at[dst_v.at[0]], dsem).wait()


_WCHUNKS = [(k * CH, CH) for k in range(RPS // CH)] + [
    ((RPS // CH) * CH, RPS - (RPS // CH) * CH)]


def _zero_slice(buf, acc_sh, base):
    _zero_rows(buf, CH, HD)
    for off, ln in _WCHUNKS:
        pltpu.sync_copy(buf.at[pl.ds(0, ln)], acc_sh.at[pl.ds(base + off, ln)])


def _writeout_slice(buf, acc_sh, base, write_fn):
    for off, ln in _WCHUNKS:
        pltpu.sync_copy(acc_sh.at[pl.ds(base + off, ln)], buf.at[pl.ds(0, ln)])
        write_fn(buf.at[pl.ds(0, ln)], off, ln)


def _sc_pass1_body(x2_hbm, src_hbm, dst_hbm, part_out, degp_out,
                   src_v, dst_v, rows0_v, rows1_v, rows2_v, rows3_v, ones_v,
                   deg_v, acc_sh, deg_sh,
                   gsem0, gsem1, gsem2, gsem3,
                   ssem0, ssem1, ssem2, ssem3, dsem):
    c = lax.axis_index("c")
    s = lax.axis_index("s")
    wid = c * NS + s
    base = s * RPS

    pltpu.sync_copy(src_hbm.at[wid], src_v)
    pltpu.sync_copy(dst_hbm.at[wid], dst_v)
    _idx_axpy(src_v, 2, 0)

    one = jnp.ones((16,), jnp.float32)

    def orow(i, _):
        ones_v[i, pl.ds(0, 16)] = one
        return 0

    lax.fori_loop(0, CH, orow, 0)

    _zero_rows(deg_v, RPS, DEGW)
    pltpu.sync_copy(deg_v, deg_sh.at[pl.ds(base, RPS)])

    rows = (rows0_v, rows1_v, rows2_v, rows3_v)
    gsems = (gsem0, gsem1, gsem2, gsem3)
    ssems = (ssem0, ssem1, ssem2, ssem3)
    for h in range(2):
        if h == 1:
            _idx_axpy(src_v, 1, 1)
        _zero_slice(rows0_v, acc_sh, base)
        plsc.subcore_barrier()
        _seg_round(x2_hbm, src_v, dst_v, rows, gsems, ssems, acc_sh,
                   (ones_v, deg_sh, dsem) if h == 0 else None)
        plsc.subcore_barrier()

        def wr1(buf_sl, off, ln, h=h):
            pltpu.sync_copy(buf_sl, part_out.at[
                c, pl.ds(base + off, ln), pl.ds(h * HD, HD)])

        _writeout_slice(rows0_v, acc_sh, base, wr1)

    pltpu.sync_copy(deg_sh.at[pl.ds(base, RPS)], deg_v)
    pltpu.sync_copy(deg_v, degp_out.at[c, pl.ds(base, RPS)])


_sc_pass1 = functools.partial(
    pl.kernel,
    out_type=(jax.ShapeDtypeStruct((NC, N, D), jnp.float32),
              jax.ShapeDtypeStruct((NC, N, DEGW), jnp.float32)),
    mesh=_MESH,
    scratch_types=[
        pltpu.VMEM((NCH, CH), jnp.int32),
        pltpu.VMEM((NCH, CH), jnp.int32),
        pltpu.VMEM((CH, HD), jnp.float32),
        pltpu.VMEM((CH, HD), jnp.float32),
        pltpu.VMEM((CH, HD), jnp.float32),
        pltpu.VMEM((CH, HD), jnp.float32),
        pltpu.VMEM((CH, DEGW), jnp.float32),
        pltpu.VMEM((RPS, DEGW), jnp.float32),
        pltpu.VMEM_SHARED((NACC, HD), jnp.float32),
        pltpu.VMEM_SHARED((NACC, DEGW), jnp.float32),
        pltpu.SemaphoreType.DMA,
        pltpu.SemaphoreType.DMA,
        pltpu.SemaphoreType.DMA,
        pltpu.SemaphoreType.DMA,
        pltpu.SemaphoreType.DMA,
        pltpu.SemaphoreType.DMA,
        pltpu.SemaphoreType.DMA,
        pltpu.SemaphoreType.DMA,
        pltpu.SemaphoreType.DMA,
    ],
    compiler_params=_SC_PARAMS,
)(_sc_pass1_body)


def _sc_pass2_body(h2_hbm, src_hbm, dst_hbm, part_out,
                   src_v, dst_v, rows0_v, rows1_v, rows2_v, rows3_v,
                   acc_sh, gsem0, gsem1, gsem2, gsem3,
                   ssem0, ssem1, ssem2, ssem3):
    c = lax.axis_index("c")
    s = lax.axis_index("s")
    wid = c * NS + s
    base = s * RPS

    pltpu.sync_copy(src_hbm.at[wid], src_v)
    pltpu.sync_copy(dst_hbm.at[wid], dst_v)
    _idx_axpy(src_v, 2, 0)

    rows = (rows0_v, rows1_v, rows2_v, rows3_v)
    gsems = (gsem0, gsem1, gsem2, gsem3)
    ssems = (ssem0, ssem1, ssem2, ssem3)
    for h in range(2):
        if h == 1:
            _idx_axpy(src_v, 1, 1)
        for t in range(3):
            _zero_slice(rows0_v, acc_sh, base)
            plsc.subcore_barrier()
            _seg_round(h2_hbm.at[t], src_v, dst_v, rows, gsems, ssems,
                       acc_sh, None)
            plsc.subcore_barrier()

            def wr2(buf_sl, off, ln, t=t, h=h):
                pltpu.sync_copy(buf_sl, part_out.at[
                    t, c, pl.ds(base + off, ln), pl.ds(h * HD, HD)])

            _writeout_slice(rows0_v, acc_sh, base, wr2)


_sc_pass2 = functools.partial(
    pl.kernel,
    out_type=jax.ShapeDtypeStruct((3, NC, N, D), jnp.float32),
    mesh=_MESH,
    scratch_types=[
        pltpu.VMEM((NCH, CH), jnp.int32),
        pltpu.VMEM((NCH, CH), jnp.int32),
        pltpu.VMEM((CH, HD), jnp.float32),
        pltpu.VMEM((CH, HD), jnp.float32),
        pltpu.VMEM((CH, HD), jnp.float32),
        pltpu.VMEM((CH, HD), jnp.float32),
        pltpu.VMEM_SHARED((NACC, HD), jnp.float32),
        pltpu.SemaphoreType.DMA,
        pltpu.SemaphoreType.DMA,
        pltpu.SemaphoreType.DMA,
        pltpu.SemaphoreType.DMA,
        pltpu.SemaphoreType.DMA,
        pltpu.SemaphoreType.DMA,
        pltpu.SemaphoreType.DMA,
        pltpu.SemaphoreType.DMA,
    ],
    compiler_params=_SC_PARAMS,
)(_sc_pass2_body)


BLK = 1000


def _inv_deg(degp):
    deg = degp[0, :, 0:1] + degp[1, :, 0:1]
    return 1.0 / jnp.maximum(deg, 1.0)


def _tc1_body(x_ref, part_ref, degp_ref, w1s_ref, w1n_ref, b1_ref, h_ref):
    x = x_ref[...]
    inv = _inv_deg(degp_ref)
    agg = (part_ref[0] + part_ref[1]) * inv
    for e in range(3):
        h = (jnp.dot(x, w1s_ref[e], preferred_element_type=jnp.float32)
             + jnp.dot(agg, w1n_ref[e], preferred_element_type=jnp.float32)
             + b1_ref[e][None, :])
        h_ref[e] = jnp.maximum(h, 0.0)


_tc1 = pl.pallas_call(
    _tc1_body,
    grid=(N // BLK,),
    in_specs=[
        pl.BlockSpec((BLK, D), lambda i: (i, 0)),
        pl.BlockSpec((NC, BLK, D), lambda i: (0, i, 0)),
        pl.BlockSpec((NC, BLK, DEGW), lambda i: (0, i, 0)),
        pl.BlockSpec((3, D, D), lambda i: (0, 0, 0)),
        pl.BlockSpec((3, D, D), lambda i: (0, 0, 0)),
        pl.BlockSpec((3, D), lambda i: (0, 0)),
    ],
    out_specs=pl.BlockSpec((3, BLK, D), lambda i: (0, i, 0)),
    out_shape=jax.ShapeDtypeStruct((3, N, D), jnp.float32),
)


def _tc2_body(h_ref, part_ref, degp_ref, w2s_ref, w2n_ref, b2_ref, o_ref):
    inv = _inv_deg(degp_ref)
    for e in range(3):
        agg = (part_ref[e, 0] + part_ref[e, 1]) * inv
        o = (jnp.dot(h_ref[e], w2s_ref[e], preferred_element_type=jnp.float32)
             + jnp.dot(agg, w2n_ref[e], preferred_element_type=jnp.float32)
             + b2_ref[e][None, :])
        if e == 1:
            n = jnp.sqrt(jnp.sum(o * o, axis=-1, keepdims=True))
            o = o / jnp.maximum(n, 1e-6)
        elif e == 2:
            n = jnp.maximum(
                jnp.sqrt(jnp.sum(o * o, axis=-1, keepdims=True)), 1e-6)
            o = jnp.tanh(n) * o / n
        o_ref[e] = o


_tc2 = pl.pallas_call(
    _tc2_body,
    grid=(N // BLK,),
    in_specs=[
        pl.BlockSpec((3, BLK, D), lambda i: (0, i, 0)),
        pl.BlockSpec((3, NC, BLK, D), lambda i: (0, 0, i, 0)),
        pl.BlockSpec((NC, BLK, DEGW), lambda i: (0, i, 0)),
        pl.BlockSpec((3, D, D), lambda i: (0, 0, 0)),
        pl.BlockSpec((3, D, D), lambda i: (0, 0, 0)),
        pl.BlockSpec((3, D), lambda i: (0, 0)),
    ],
    out_specs=pl.BlockSpec((3, BLK, D), lambda i: (0, i, 0)),
    out_shape=jax.ShapeDtypeStruct((3, N, D), jnp.float32),
)


def kernel(node_features, edge_index_list, target_node_idx,
           euc_W1s, euc_W1n, euc_b1, euc_W2s, euc_W2n, euc_b2,
           sph_W1s, sph_W1n, sph_b1, sph_W2s, sph_W2n, sph_b2,
           hyp_W1s, hyp_W1n, hyp_b1, hyp_W2s, hyp_W2n, hyp_b2):
    x = node_features
    npad = EPAD - E
    src = jnp.concatenate(
        [edge_index_list[0],
         jnp.zeros((npad,), jnp.int32)]).reshape(NW, NCH, CH)
    dst = jnp.concatenate(
        [edge_index_list[1],
         jnp.full((npad,), N, jnp.int32)]).reshape(NW, NCH, CH)

    part1, degp = _sc_pass1(x.reshape(2 * N, HD), src, dst)

    w1s = jnp.stack([euc_W1s, sph_W1s, hyp_W1s])
    w1n = jnp.stack([euc_W1n, sph_W1n, hyp_W1n])
    b1 = jnp.stack([euc_b1, sph_b1, hyp_b1])
    h = _tc1(x, part1, degp, w1s, w1n, b1)

    part2 = _sc_pass2(h.reshape(3, 2 * N, HD), src, dst)

    w2s = jnp.stack([euc_W2s, sph_W2s, hyp_W2s])
    w2n = jnp.stack([euc_W2n, sph_W2n, hyp_W2n])
    b2 = jnp.stack([euc_b2, sph_b2, hyp_b2])
    return _tc2(h, part2, degp, w2s, w2n, b2)


# speedup vs baseline: 1.7568x; 1.7568x over previous
import functools

import jax
import jax.numpy as jnp
from jax import lax
from jax.experimental import pallas as pl
from jax.experimental.pallas import tpu as pltpu
from jax.experimental.pallas import tpu_sc as plsc

N = 10000
D = 128
HD = 64
E = 320000
NC = 2
NS = 16
NW = NC * NS
CH = 96
NCH = 105
EPW = NCH * CH
EPAD = NW * EPW
RPS = N // NS
DEGW = 16
NACC = N + 8

_MESH = plsc.VectorSubcoreMesh(core_axis_name="c", subcore_axis_name="s",
                               num_cores=NC, num_subcores=NS)
_SC_PARAMS = pltpu.CompilerParams(use_tc_tiling_on_sc=False)


def _zero_rows(ref, nrows, width):
    z = jnp.zeros((16,), jnp.float32)

    def row(i, _):
        for j in range(width // 16):
            ref[i, pl.ds(j * 16, 16)] = z
        return 0

    lax.fori_loop(0, nrows, row, 0)


def _idx_axpy(ref, mul, add):
    def row(i, _):
        for j in range(CH // 16):
            sl = pl.ds(j * 16, 16)
            ref[i, sl] = ref[i, sl] * mul + add
        return 0

    lax.fori_loop(0, NCH, row, 0)


NBUF = 4


def _seg_round(tab, src_v, dst_v, rows, gsems, ssems, acc_sh, deg_tup):
    def g_start(idx, b):
        pltpu.async_copy(tab.at[src_v.at[idx]], rows[b], gsems[b])

    def g_wait(idx, b):
        pltpu.make_async_copy(tab.at[src_v.at[idx]], rows[b], gsems[b]).wait()

    def s_wait(b):
        pltpu.make_async_copy(
            rows[b], acc_sh.at[dst_v.at[0]], ssems[b]).wait()

    def block(idx, b, wait_s, issue_g):
        g_wait(idx, b)
        pltpu.async_copy(rows[b], acc_sh.at[dst_v.at[idx]], ssems[b],
                         add=True)
        if deg_tup is not None:
            ones_v, deg_sh, dsem = deg_tup
            pltpu.async_copy(ones_v, deg_sh.at[dst_v.at[idx]], dsem,
                             add=True)
            if wait_s:
                pltpu.make_async_copy(
                    ones_v, deg_sh.at[dst_v.at[0]], dsem).wait()
        if wait_s:
            s_wait((b - 2) % NBUF)
        if issue_g:
            g_start(idx + 2, (b + 2) % NBUF)

    g_start(0, 0)
    g_start(1, 1)
    block(0, 0, False, True)
    block(1, 1, False, True)

    def quad(j, _):
        i = 4 * j + 2
        for b in (2, 3, 0, 1):
            block(i, b, True, True)
            i = i + 1
        return 0

    nquads = (NCH - 4) // 4
    lax.fori_loop(0, nquads, quad, 0)
    for i in range(2 + 4 * nquads, NCH):
        block(i, i % NBUF, True, i + 2 <= NCH - 1)
    s_wait((NCH - 2) % NBUF)
    s_wait((NCH - 1) % NBUF)
    if deg_tup is not None:
        ones_v, deg_sh, dsem = deg_tup
        for _ in range(2):
            pltpu.make_async_copy(
                ones_v, deg_sh.at[dst_v.at[0]], dsem).wait()


_WCHUNKS = [(k * CH, CH) for k in range(RPS // CH)] + [
    ((RPS // CH) * CH, RPS - (RPS // CH) * CH)]


def _zero_slice(buf, acc_sh, base):
    _zero_rows(buf, CH, HD)
    for off, ln in _WCHUNKS:
        pltpu.sync_copy(buf.at[pl.ds(0, ln)], acc_sh.at[pl.ds(base + off, ln)])


def _writeout_slice(buf, acc_sh, base, write_fn):
    for off, ln in _WCHUNKS:
        pltpu.sync_copy(acc_sh.at[pl.ds(base + off, ln)], buf.at[pl.ds(0, ln)])
        write_fn(buf.at[pl.ds(0, ln)], off, ln)


def _sc_pass1_body(x2_hbm, src_hbm, dst_hbm, part_out, degp_out,
                   src_v, dst_v, rows0_v, rows1_v, rows2_v, rows3_v, ones_v,
                   deg_v, acc_sh, deg_sh,
                   gsem0, gsem1, gsem2, gsem3,
                   ssem0, ssem1, ssem2, ssem3, dsem):
    c = lax.axis_index("c")
    s = lax.axis_index("s")
    wid = c * NS + s
    base = s * RPS

    pltpu.sync_copy(src_hbm.at[wid], src_v)
    pltpu.sync_copy(dst_hbm.at[wid], dst_v)
    _idx_axpy(src_v, 2, 0)

    one = jnp.ones((16,), jnp.float32)

    def orow(i, _):
        ones_v[i, pl.ds(0, 16)] = one
        return 0

    lax.fori_loop(0, CH, orow, 0)

    _zero_rows(deg_v, RPS, DEGW)
    pltpu.sync_copy(deg_v, deg_sh.at[pl.ds(base, RPS)])

    rows = (rows0_v, rows1_v, rows2_v, rows3_v)
    gsems = (gsem0, gsem1, gsem2, gsem3)
    ssems = (ssem0, ssem1, ssem2, ssem3)
    for h in range(2):
        if h == 1:
            _idx_axpy(src_v, 1, 1)
        _zero_slice(rows0_v, acc_sh, base)
        plsc.subcore_barrier()
        _seg_round(x2_hbm, src_v, dst_v, rows, gsems, ssems, acc_sh,
                   (ones_v, deg_sh, dsem) if h == 0 else None)
        plsc.subcore_barrier()

        def wr1(buf_sl, off, ln, h=h):
            pltpu.sync_copy(buf_sl, part_out.at[
                c, pl.ds(base + off, ln), pl.ds(h * HD, HD)])

        _writeout_slice(rows0_v, acc_sh, base, wr1)

    pltpu.sync_copy(deg_sh.at[pl.ds(base, RPS)], deg_v)
    pltpu.sync_copy(deg_v, degp_out.at[c, pl.ds(base, RPS)])


_sc_pass1 = functools.partial(
    pl.kernel,
    out_type=(jax.ShapeDtypeStruct((NC, N, D), jnp.float32),
              jax.ShapeDtypeStruct((NC, N, DEGW), jnp.float32)),
    mesh=_MESH,
    scratch_types=[
        pltpu.VMEM((NCH, CH), jnp.int32),
        pltpu.VMEM((NCH, CH), jnp.int32),
        pltpu.VMEM((CH, HD), jnp.float32),
        pltpu.VMEM((CH, HD), jnp.float32),
        pltpu.VMEM((CH, HD), jnp.float32),
        pltpu.VMEM((CH, HD), jnp.float32),
        pltpu.VMEM((CH, DEGW), jnp.float32),
        pltpu.VMEM((RPS, DEGW), jnp.float32),
        pltpu.VMEM_SHARED((NACC, HD), jnp.float32),
        pltpu.VMEM_SHARED((NACC, DEGW), jnp.float32),
        pltpu.SemaphoreType.DMA,
        pltpu.SemaphoreType.DMA,
        pltpu.SemaphoreType.DMA,
        pltpu.SemaphoreType.DMA,
        pltpu.SemaphoreType.DMA,
        pltpu.SemaphoreType.DMA,
        pltpu.SemaphoreType.DMA,
        pltpu.SemaphoreType.DMA,
        pltpu.SemaphoreType.DMA,
    ],
    compiler_params=_SC_PARAMS,
)(_sc_pass1_body)


def _sc_pass2_body(h2_hbm, src_hbm, dst_hbm, part_out,
                   src_v, dst_v, rows0_v, rows1_v, rows2_v, rows3_v,
                   acc_sh, gsem0, gsem1, gsem2, gsem3,
                   ssem0, ssem1, ssem2, ssem3):
    c = lax.axis_index("c")
    s = lax.axis_index("s")
    wid = c * NS + s
    base = s * RPS

    pltpu.sync_copy(src_hbm.at[wid], src_v)
    pltpu.sync_copy(dst_hbm.at[wid], dst_v)
    _idx_axpy(src_v, 2, 0)

    rows = (rows0_v, rows1_v, rows2_v, rows3_v)
    gsems = (gsem0, gsem1, gsem2, gsem3)
    ssems = (ssem0, ssem1, ssem2, ssem3)
    for h in range(2):
        if h == 1:
            _idx_axpy(src_v, 1, 1)
        for t in range(3):
            _zero_slice(rows0_v, acc_sh, base)
            plsc.subcore_barrier()
            _seg_round(h2_hbm.at[t], src_v, dst_v, rows, gsems, ssems,
                       acc_sh, None)
            plsc.subcore_barrier()

            def wr2(buf_sl, off, ln, t=t, h=h):
                pltpu.sync_copy(buf_sl, part_out.at[
                    t, c, pl.ds(base + off, ln), pl.ds(h * HD, HD)])

            _writeout_slice(rows0_v, acc_sh, base, wr2)


_sc_pass2 = functools.partial(
    pl.kernel,
    out_type=jax.ShapeDtypeStruct((3, NC, N, D), jnp.float32),
    mesh=_MESH,
    scratch_types=[
        pltpu.VMEM((NCH, CH), jnp.int32),
        pltpu.VMEM((NCH, CH), jnp.int32),
        pltpu.VMEM((CH, HD), jnp.float32),
        pltpu.VMEM((CH, HD), jnp.float32),
        pltpu.VMEM((CH, HD), jnp.float32),
        pltpu.VMEM((CH, HD), jnp.float32),
        pltpu.VMEM_SHARED((NACC, HD), jnp.float32),
        pltpu.SemaphoreType.DMA,
        pltpu.SemaphoreType.DMA,
        pltpu.SemaphoreType.DMA,
        pltpu.SemaphoreType.DMA,
        pltpu.SemaphoreType.DMA,
        pltpu.SemaphoreType.DMA,
        pltpu.SemaphoreType.DMA,
        pltpu.SemaphoreType.DMA,
    ],
    compiler_params=_SC_PARAMS,
)(_sc_pass2_body)


BLK = 1000


def _inv_deg(degp):
    deg = degp[0, :, 0:1] + degp[1, :, 0:1]
    return 1.0 / jnp.maximum(deg, 1.0)


def _tc1_body(x_ref, part_ref, degp_ref, w1s_ref, w1n_ref, b1_ref, h_ref):
    x = x_ref[...]
    inv = _inv_deg(degp_ref)
    agg = (part_ref[0] + part_ref[1]) * inv
    for e in range(3):
        h = (jnp.dot(x, w1s_ref[e], preferred_element_type=jnp.float32)
             + jnp.dot(agg, w1n_ref[e], preferred_element_type=jnp.float32)
             + b1_ref[e][None, :])
        h_ref[e] = jnp.maximum(h, 0.0)


_tc1 = pl.pallas_call(
    _tc1_body,
    grid=(N // BLK,),
    in_specs=[
        pl.BlockSpec((BLK, D), lambda i: (i, 0)),
        pl.BlockSpec((NC, BLK, D), lambda i: (0, i, 0)),
        pl.BlockSpec((NC, BLK, DEGW), lambda i: (0, i, 0)),
        pl.BlockSpec((3, D, D), lambda i: (0, 0, 0)),
        pl.BlockSpec((3, D, D), lambda i: (0, 0, 0)),
        pl.BlockSpec((3, D), lambda i: (0, 0)),
    ],
    out_specs=pl.BlockSpec((3, BLK, D), lambda i: (0, i, 0)),
    out_shape=jax.ShapeDtypeStruct((3, N, D), jnp.float32),
)


def _tc2_body(h_ref, part_ref, degp_ref, w2s_ref, w2n_ref, b2_ref, o_ref):
    inv = _inv_deg(degp_ref)
    for e in range(3):
        agg = (part_ref[e, 0] + part_ref[e, 1]) * inv
        o = (jnp.dot(h_ref[e], w2s_ref[e], preferred_element_type=jnp.float32)
             + jnp.dot(agg, w2n_ref[e], preferred_element_type=jnp.float32)
             + b2_ref[e][None, :])
        if e == 1:
            n = jnp.sqrt(jnp.sum(o * o, axis=-1, keepdims=True))
            o = o / jnp.maximum(n, 1e-6)
        elif e == 2:
            n = jnp.maximum(
                jnp.sqrt(jnp.sum(o * o, axis=-1, keepdims=True)), 1e-6)
            o = jnp.tanh(n) * o / n
        o_ref[e] = o


_tc2 = pl.pallas_call(
    _tc2_body,
    grid=(N // BLK,),
    in_specs=[
        pl.BlockSpec((3, BLK, D), lambda i: (0, i, 0)),
        pl.BlockSpec((3, NC, BLK, D), lambda i: (0, 0, i, 0)),
        pl.BlockSpec((NC, BLK, DEGW), lambda i: (0, i, 0)),
        pl.BlockSpec((3, D, D), lambda i: (0, 0, 0)),
        pl.BlockSpec((3, D, D), lambda i: (0, 0, 0)),
        pl.BlockSpec((3, D), lambda i: (0, 0)),
    ],
    out_specs=pl.BlockSpec((3, BLK, D), lambda i: (0, i, 0)),
    out_shape=jax.ShapeDtypeStruct((3, N, D), jnp.float32),
)


def kernel(node_features, edge_index_list, target_node_idx,
           euc_W1s, euc_W1n, euc_b1, euc_W2s, euc_W2n, euc_b2,
           sph_W1s, sph_W1n, sph_b1, sph_W2s, sph_W2n, sph_b2,
           hyp_W1s, hyp_W1n, hyp_b1, hyp_W2s, hyp_W2n, hyp_b2):
    x = node_features
    npad = EPAD - E
    src = jnp.concatenate(
        [edge_index_list[0],
         jnp.zeros((npad,), jnp.int32)]).reshape(NW, NCH, CH)
    dst = jnp.concatenate(
        [edge_index_list[1],
         jnp.full((npad,), N, jnp.int32)]).reshape(NW, NCH, CH)

    part1, degp = _sc_pass1(x.reshape(2 * N, HD), src, dst)

    w1s = jnp.stack([euc_W1s, sph_W1s, hyp_W1s])
    w1n = jnp.stack([euc_W1n, sph_W1n, hyp_W1n])
    b1 = jnp.stack([euc_b1, sph_b1, hyp_b1])
    h = _tc1(x, part1, degp, w1s, w1n, b1)

    part2 = _sc_pass2(h.reshape(3, 2 * N, HD), src, dst)

    w2s = jnp.stack([euc_W2s, sph_W2s, hyp_W2s])
    w2n = jnp.stack([euc_W2n, sph_W2n, hyp_W2n])
    b2 = jnp.stack([euc_b2, sph_b2, hyp_b2])
    return _tc2(h, part2, degp, w2s, w2n, b2)


# speedup vs baseline: 1.9991x; 1.1379x over previous
import functools

import jax
import jax.numpy as jnp
from jax import lax
from jax.experimental import pallas as pl
from jax.experimental.pallas import tpu as pltpu
from jax.experimental.pallas import tpu_sc as plsc

N = 10000
D = 128
HD = 64
E = 320000
NC = 2
NS = 16
NW = NC * NS
CH = 64
NCH = 157
EPW = NCH * CH
EPAD = NW * EPW
RPS = N // NS
DEGW = 16
NACC = N + 8

_MESH = plsc.VectorSubcoreMesh(core_axis_name="c", subcore_axis_name="s",
                               num_cores=NC, num_subcores=NS)
_SC_PARAMS = pltpu.CompilerParams(use_tc_tiling_on_sc=False)


def _zero_rows(ref, nrows, width):
    z = jnp.zeros((16,), jnp.float32)

    def row(i, _):
        for j in range(width // 16):
            ref[i, pl.ds(j * 16, 16)] = z
        return 0

    lax.fori_loop(0, nrows, row, 0)


def _idx_axpy(ref, mul, add):
    def row(i, _):
        for j in range(CH // 16):
            sl = pl.ds(j * 16, 16)
            ref[i, sl] = ref[i, sl] * mul + add
        return 0

    lax.fori_loop(0, NCH, row, 0)


NBUF = 4


def _seg_round(tab, src_v, dst_v, rows, gsems, ssems, acc_sh, deg_tup):
    def g_start(idx, b):
        pltpu.async_copy(tab.at[src_v.at[idx]], rows[b], gsems[b])

    def g_wait(idx, b):
        pltpu.make_async_copy(tab.at[src_v.at[idx]], rows[b], gsems[b]).wait()

    def s_wait(b):
        pltpu.make_async_copy(
            rows[b], acc_sh.at[dst_v.at[0]], ssems[b]).wait()

    def block(idx, b, wait_s, issue_g):
        g_wait(idx, b)
        pltpu.async_copy(rows[b], acc_sh.at[dst_v.at[idx]], ssems[b],
                         add=True)
        if deg_tup is not None:
            ones_v, deg_sh, dsem = deg_tup
            pltpu.async_copy(ones_v, deg_sh.at[dst_v.at[idx]], dsem,
                             add=True)
            if wait_s:
                pltpu.make_async_copy(
                    ones_v, deg_sh.at[dst_v.at[0]], dsem).wait()
        if wait_s:
            s_wait((b - 2) % NBUF)
        if issue_g:
            g_start(idx + 2, (b + 2) % NBUF)

    g_start(0, 0)
    g_start(1, 1)
    block(0, 0, False, True)
    block(1, 1, False, True)

    def quad(j, _):
        i = 4 * j + 2
        for b in (2, 3, 0, 1):
            block(i, b, True, True)
            i = i + 1
        return 0

    nquads = (NCH - 4) // 4
    lax.fori_loop(0, nquads, quad, 0)
    for i in range(2 + 4 * nquads, NCH):
        block(i, i % NBUF, True, i + 2 <= NCH - 1)
    s_wait((NCH - 2) % NBUF)
    s_wait((NCH - 1) % NBUF)
    if deg_tup is not None:
        ones_v, deg_sh, dsem = deg_tup
        for _ in range(2):
            pltpu.make_async_copy(
                ones_v, deg_sh.at[dst_v.at[0]], dsem).wait()


_WCHUNKS = [(k * CH, CH) for k in range(RPS // CH)] + [
    ((RPS // CH) * CH, RPS - (RPS // CH) * CH)]


def _zero_slice(buf, acc_sh, base):
    _zero_rows(buf, CH, HD)
    for off, ln in _WCHUNKS:
        pltpu.sync_copy(buf.at[pl.ds(0, ln)], acc_sh.at[pl.ds(base + off, ln)])


def _writeout_slice(buf, acc_sh, base, write_fn):
    for off, ln in _WCHUNKS:
        pltpu.sync_copy(acc_sh.at[pl.ds(base + off, ln)], buf.at[pl.ds(0, ln)])
        write_fn(buf.at[pl.ds(0, ln)], off, ln)


def _sc_pass1_body(x2_hbm, src_hbm, dst_hbm, part_out, degp_out,
                   src_v, dst_v, rows0_v, rows1_v, rows2_v, rows3_v, ones_v,
                   deg_v, acc_sh, deg_sh,
                   gsem0, gsem1, gsem2, gsem3,
                   ssem0, ssem1, ssem2, ssem3, dsem):
    c = lax.axis_index("c")
    s = lax.axis_index("s")
    wid = c * NS + s
    base = s * RPS

    pltpu.sync_copy(src_hbm.at[wid], src_v)
    pltpu.sync_copy(dst_hbm.at[wid], dst_v)
    _idx_axpy(src_v, 2, 0)

    one = jnp.ones((16,), jnp.float32)

    def orow(i, _):
        ones_v[i, pl.ds(0, 16)] = one
        return 0

    lax.fori_loop(0, CH, orow, 0)

    _zero_rows(deg_v, RPS, DEGW)
    pltpu.sync_copy(deg_v, deg_sh.at[pl.ds(base, RPS)])

    rows = (rows0_v, rows1_v, rows2_v, rows3_v)
    gsems = (gsem0, gsem1, gsem2, gsem3)
    ssems = (ssem0, ssem1, ssem2, ssem3)
    for h in range(2):
        if h == 1:
            _idx_axpy(src_v, 1, 1)
        _zero_slice(rows0_v, acc_sh, base)
        plsc.subcore_barrier()
        _seg_round(x2_hbm, src_v, dst_v, rows, gsems, ssems, acc_sh,
                   (ones_v, deg_sh, dsem) if h == 0 else None)
        plsc.subcore_barrier()

        def wr1(buf_sl, off, ln, h=h):
            pltpu.sync_copy(buf_sl, part_out.at[
                c, pl.ds(base + off, ln), pl.ds(h * HD, HD)])

        _writeout_slice(rows0_v, acc_sh, base, wr1)

    pltpu.sync_copy(deg_sh.at[pl.ds(base, RPS)], deg_v)
    pltpu.sync_copy(deg_v, degp_out.at[c, pl.ds(base, RPS)])


_sc_pass1 = functools.partial(
    pl.kernel,
    out_type=(jax.ShapeDtypeStruct((NC, N, D), jnp.float32),
              jax.ShapeDtypeStruct((NC, N, DEGW), jnp.float32)),
    mesh=_MESH,
    scratch_types=[
        pltpu.VMEM((NCH, CH), jnp.int32),
        pltpu.VMEM((NCH, CH), jnp.int32),
        pltpu.VMEM((CH, HD), jnp.float32),
        pltpu.VMEM((CH, HD), jnp.float32),
        pltpu.VMEM((CH, HD), jnp.float32),
        pltpu.VMEM((CH, HD), jnp.float32),
        pltpu.VMEM((CH, DEGW), jnp.float32),
        pltpu.VMEM((RPS, DEGW), jnp.float32),
        pltpu.VMEM_SHARED((NACC, HD), jnp.float32),
        pltpu.VMEM_SHARED((NACC, DEGW), jnp.float32),
        pltpu.SemaphoreType.DMA,
        pltpu.SemaphoreType.DMA,
        pltpu.SemaphoreType.DMA,
        pltpu.SemaphoreType.DMA,
        pltpu.SemaphoreType.DMA,
        pltpu.SemaphoreType.DMA,
        pltpu.SemaphoreType.DMA,
        pltpu.SemaphoreType.DMA,
        pltpu.SemaphoreType.DMA,
    ],
    compiler_params=_SC_PARAMS,
)(_sc_pass1_body)


def _sc_pass2_body(h2_hbm, src_hbm, dst_hbm, part_out,
                   src_v, dst_v, rows0_v, rows1_v, rows2_v, rows3_v,
                   acc_sh, gsem0, gsem1, gsem2, gsem3,
                   ssem0, ssem1, ssem2, ssem3):
    c = lax.axis_index("c")
    s = lax.axis_index("s")
    wid = c * NS + s
    base = s * RPS

    pltpu.sync_copy(src_hbm.at[wid], src_v)
    pltpu.sync_copy(dst_hbm.at[wid], dst_v)
    _idx_axpy(src_v, 2, 0)

    rows = (rows0_v, rows1_v, rows2_v, rows3_v)
    gsems = (gsem0, gsem1, gsem2, gsem3)
    ssems = (ssem0, ssem1, ssem2, ssem3)
    for h in range(2):
        if h == 1:
            _idx_axpy(src_v, 1, 1)
        for t in range(3):
            _zero_slice(rows0_v, acc_sh, base)
            plsc.subcore_barrier()
            _seg_round(h2_hbm.at[t], src_v, dst_v, rows, gsems, ssems,
                       acc_sh, None)
            plsc.subcore_barrier()

            def wr2(buf_sl, off, ln, t=t, h=h):
                pltpu.sync_copy(buf_sl, part_out.at[
                    t, c, pl.ds(base + off, ln), pl.ds(h * HD, HD)])

            _writeout_slice(rows0_v, acc_sh, base, wr2)


_sc_pass2 = functools.partial(
    pl.kernel,
    out_type=jax.ShapeDtypeStruct((3, NC, N, D), jnp.float32),
    mesh=_MESH,
    scratch_types=[
        pltpu.VMEM((NCH, CH), jnp.int32),
        pltpu.VMEM((NCH, CH), jnp.int32),
        pltpu.VMEM((CH, HD), jnp.float32),
        pltpu.VMEM((CH, HD), jnp.float32),
        pltpu.VMEM((CH, HD), jnp.float32),
        pltpu.VMEM((CH, HD), jnp.float32),
        pltpu.VMEM_SHARED((NACC, HD), jnp.float32),
        pltpu.SemaphoreType.DMA,
        pltpu.SemaphoreType.DMA,
        pltpu.SemaphoreType.DMA,
        pltpu.SemaphoreType.DMA,
        pltpu.SemaphoreType.DMA,
        pltpu.SemaphoreType.DMA,
        pltpu.SemaphoreType.DMA,
        pltpu.SemaphoreType.DMA,
    ],
    compiler_params=_SC_PARAMS,
)(_sc_pass2_body)


BLK = 1000


def _inv_deg(degp):
    deg = degp[0, :, 0:1] + degp[1, :, 0:1]
    return 1.0 / jnp.maximum(deg, 1.0)


def _tc1_body(x_ref, part_ref, degp_ref, w1s_ref, w1n_ref, b1_ref, h_ref):
    x = x_ref[...]
    inv = _inv_deg(degp_ref)
    agg = (part_ref[0] + part_ref[1]) * inv
    for e in range(3):
        h = (jnp.dot(x, w1s_ref[e], preferred_element_type=jnp.float32)
             + jnp.dot(agg, w1n_ref[e], preferred_element_type=jnp.float32)
             + b1_ref[e][None, :])
        h_ref[e] = jnp.maximum(h, 0.0)


_tc1 = pl.pallas_call(
    _tc1_body,
    grid=(N // BLK,),
    in_specs=[
        pl.BlockSpec((BLK, D), lambda i: (i, 0)),
        pl.BlockSpec((NC, BLK, D), lambda i: (0, i, 0)),
        pl.BlockSpec((NC, BLK, DEGW), lambda i: (0, i, 0)),
        pl.BlockSpec((3, D, D), lambda i: (0, 0, 0)),
        pl.BlockSpec((3, D, D), lambda i: (0, 0, 0)),
        pl.BlockSpec((3, D), lambda i: (0, 0)),
    ],
    out_specs=pl.BlockSpec((3, BLK, D), lambda i: (0, i, 0)),
    out_shape=jax.ShapeDtypeStruct((3, N, D), jnp.float32),
)


def _tc2_body(h_ref, part_ref, degp_ref, w2s_ref, w2n_ref, b2_ref, o_ref):
    inv = _inv_deg(degp_ref)
    for e in range(3):
        agg = (part_ref[e, 0] + part_ref[e, 1]) * inv
        o = (jnp.dot(h_ref[e], w2s_ref[e], preferred_element_type=jnp.float32)
             + jnp.dot(agg, w2n_ref[e], preferred_element_type=jnp.float32)
             + b2_ref[e][None, :])
        if e == 1:
            n = jnp.sqrt(jnp.sum(o * o, axis=-1, keepdims=True))
            o = o / jnp.maximum(n, 1e-6)
        elif e == 2:
            n = jnp.maximum(
                jnp.sqrt(jnp.sum(o * o, axis=-1, keepdims=True)), 1e-6)
            o = jnp.tanh(n) * o / n
        o_ref[e] = o


_tc2 = pl.pallas_call(
    _tc2_body,
    grid=(N // BLK,),
    in_specs=[
        pl.BlockSpec((3, BLK, D), lambda i: (0, i, 0)),
        pl.BlockSpec((3, NC, BLK, D), lambda i: (0, 0, i, 0)),
        pl.BlockSpec((NC, BLK, DEGW), lambda i: (0, i, 0)),
        pl.BlockSpec((3, D, D), lambda i: (0, 0, 0)),
        pl.BlockSpec((3, D, D), lambda i: (0, 0, 0)),
        pl.BlockSpec((3, D), lambda i: (0, 0)),
    ],
    out_specs=pl.BlockSpec((3, BLK, D), lambda i: (0, i, 0)),
    out_shape=jax.ShapeDtypeStruct((3, N, D), jnp.float32),
)


def kernel(node_features, edge_index_list, target_node_idx,
           euc_W1s, euc_W1n, euc_b1, euc_W2s, euc_W2n, euc_b2,
           sph_W1s, sph_W1n, sph_b1, sph_W2s, sph_W2n, sph_b2,
           hyp_W1s, hyp_W1n, hyp_b1, hyp_W2s, hyp_W2n, hyp_b2):
    x = node_features
    npad = EPAD - E
    src = jnp.concatenate(
        [edge_index_list[0],
         jnp.zeros((npad,), jnp.int32)]).reshape(NW, NCH, CH)
    dst = jnp.concatenate(
        [edge_index_list[1],
         jnp.full((npad,), N, jnp.int32)]).reshape(NW, NCH, CH)

    part1, degp = _sc_pass1(x.reshape(2 * N, HD), src, dst)

    w1s = jnp.stack([euc_W1s, sph_W1s, hyp_W1s])
    w1n = jnp.stack([euc_W1n, sph_W1n, hyp_W1n])
    b1 = jnp.stack([euc_b1, sph_b1, hyp_b1])
    h = _tc1(x, part1, degp, w1s, w1n, b1)

    part2 = _sc_pass2(h.reshape(3, 2 * N, HD), src, dst)

    w2s = jnp.stack([euc_W2s, sph_W2s, hyp_W2s])
    w2n = jnp.stack([euc_W2n, sph_W2n, hyp_W2n])
    b2 = jnp.stack([euc_b2, sph_b2, hyp_b2])
    return _tc2(h, part2, degp, w2s, w2n, b2)


# speedup vs baseline: 2.9277x; 1.4645x over previous
import functools

import jax
import jax.numpy as jnp
from jax import lax
from jax.experimental import pallas as pl
from jax.experimental.pallas import tpu as pltpu
from jax.experimental.pallas import tpu_sc as plsc

N = 10000
D = 128
HD = 64
E = 320000
NC = 2
NS = 16
NW = NC * NS
CH = 80
NCH = 125
EPW = NCH * CH
EPAD = NW * EPW
RPS = N // NS
DEGW = 16
NACC = N + 8

_MESH = plsc.VectorSubcoreMesh(core_axis_name="c", subcore_axis_name="s",
                               num_cores=NC, num_subcores=NS)
_SC_PARAMS = pltpu.CompilerParams(use_tc_tiling_on_sc=False)


def _zero_rows(ref, nrows, width):
    z = jnp.zeros((16,), jnp.float32)

    def row(i, _):
        for j in range(width // 16):
            ref[i, pl.ds(j * 16, 16)] = z
        return 0

    lax.fori_loop(0, nrows, row, 0)


def _idx_axpy(ref, mul, add):
    def row(i, _):
        for j in range(CH // 16):
            sl = pl.ds(j * 16, 16)
            ref[i, sl] = ref[i, sl] * mul + add
        return 0

    lax.fori_loop(0, NCH, row, 0)


NBUF = 4


def _seg_round(tab, src_v, dst_v, rows, gsems, ssems, acc_sh, deg_tup):
    def g_start(idx, b):
        pltpu.async_copy(tab.at[src_v.at[idx]], rows[b], gsems[b])

    def g_wait(idx, b):
        pltpu.make_async_copy(tab.at[src_v.at[idx]], rows[b], gsems[b]).wait()

    def s_wait(b):
        pltpu.make_async_copy(
            rows[b], acc_sh.at[dst_v.at[0]], ssems[b]).wait()

    def block(idx, b, wait_s, issue_g):
        g_wait(idx, b)
        pltpu.async_copy(rows[b], acc_sh.at[dst_v.at[idx]], ssems[b],
                         add=True)
        if deg_tup is not None:
            ones_v, deg_sh, dsem = deg_tup
            pltpu.async_copy(ones_v, deg_sh.at[dst_v.at[idx]], dsem,
                             add=True)
            if wait_s:
                pltpu.make_async_copy(
                    ones_v, deg_sh.at[dst_v.at[0]], dsem).wait()
        if wait_s:
            s_wait((b - 2) % NBUF)
        if issue_g:
            g_start(idx + 2, (b + 2) % NBUF)

    g_start(0, 0)
    g_start(1, 1)
    block(0, 0, False, True)
    block(1, 1, False, True)

    def quad(j, _):
        i = 4 * j + 2
        for b in (2, 3, 0, 1):
            block(i, b, True, True)
            i = i + 1
        return 0

    nquads = (NCH - 4) // 4
    lax.fori_loop(0, nquads, quad, 0)
    for i in range(2 + 4 * nquads, NCH):
        block(i, i % NBUF, True, i + 2 <= NCH - 1)
    s_wait((NCH - 2) % NBUF)
    s_wait((NCH - 1) % NBUF)
    if deg_tup is not None:
        ones_v, deg_sh, dsem = deg_tup
        for _ in range(2):
            pltpu.make_async_copy(
                ones_v, deg_sh.at[dst_v.at[0]], dsem).wait()


_WCHUNKS = [(k * CH, CH) for k in range(RPS // CH)] + [
    ((RPS // CH) * CH, RPS - (RPS // CH) * CH)]


def _zero_slice(buf, acc_sh, base):
    _zero_rows(buf, CH, HD)
    for off, ln in _WCHUNKS:
        pltpu.sync_copy(buf.at[pl.ds(0, ln)], acc_sh.at[pl.ds(base + off, ln)])


def _writeout_slice(buf, acc_sh, base, write_fn):
    for off, ln in _WCHUNKS:
        pltpu.sync_copy(acc_sh.at[pl.ds(base + off, ln)], buf.at[pl.ds(0, ln)])
        write_fn(buf.at[pl.ds(0, ln)], off, ln)


def _sc_pass1_body(x2_hbm, src_hbm, dst_hbm, part_out, degp_out,
                   src_v, dst_v, rows0_v, rows1_v, rows2_v, rows3_v, ones_v,
                   deg_v, acc_sh, deg_sh,
                   gsem0, gsem1, gsem2, gsem3,
                   ssem0, ssem1, ssem2, ssem3, dsem):
    c = lax.axis_index("c")
    s = lax.axis_index("s")
    wid = c * NS + s
    base = s * RPS

    pltpu.sync_copy(src_hbm.at[wid], src_v)
    pltpu.sync_copy(dst_hbm.at[wid], dst_v)
    _idx_axpy(src_v, 2, 0)

    one = jnp.ones((16,), jnp.float32)

    def orow(i, _):
        ones_v[i, pl.ds(0, 16)] = one
        return 0

    lax.fori_loop(0, CH, orow, 0)

    _zero_rows(deg_v, RPS, DEGW)
    pltpu.sync_copy(deg_v, deg_sh.at[pl.ds(base, RPS)])

    rows = (rows0_v, rows1_v, rows2_v, rows3_v)
    gsems = (gsem0, gsem1, gsem2, gsem3)
    ssems = (ssem0, ssem1, ssem2, ssem3)
    for h in range(2):
        if h == 1:
            _idx_axpy(src_v, 1, 1)
        _zero_slice(rows0_v, acc_sh, base)
        plsc.subcore_barrier()
        _seg_round(x2_hbm, src_v, dst_v, rows, gsems, ssems, acc_sh,
                   (ones_v, deg_sh, dsem) if h == 0 else None)
        plsc.subcore_barrier()

        def wr1(buf_sl, off, ln, h=h):
            pltpu.sync_copy(buf_sl, part_out.at[
                c, pl.ds(base + off, ln), pl.ds(h * HD, HD)])

        _writeout_slice(rows0_v, acc_sh, base, wr1)

    pltpu.sync_copy(deg_sh.at[pl.ds(base, RPS)], deg_v)
    pltpu.sync_copy(deg_v, degp_out.at[c, pl.ds(base, RPS)])


_sc_pass1 = functools.partial(
    pl.kernel,
    out_type=(jax.ShapeDtypeStruct((NC, N, D), jnp.float32),
              jax.ShapeDtypeStruct((NC, N, DEGW), jnp.float32)),
    mesh=_MESH,
    scratch_types=[
        pltpu.VMEM((NCH, CH), jnp.int32),
        pltpu.VMEM((NCH, CH), jnp.int32),
        pltpu.VMEM((CH, HD), jnp.float32),
        pltpu.VMEM((CH, HD), jnp.float32),
        pltpu.VMEM((CH, HD), jnp.float32),
        pltpu.VMEM((CH, HD), jnp.float32),
        pltpu.VMEM((CH, DEGW), jnp.float32),
        pltpu.VMEM((RPS, DEGW), jnp.float32),
        pltpu.VMEM_SHARED((NACC, HD), jnp.float32),
        pltpu.VMEM_SHARED((NACC, DEGW), jnp.float32),
        pltpu.SemaphoreType.DMA,
        pltpu.SemaphoreType.DMA,
        pltpu.SemaphoreType.DMA,
        pltpu.SemaphoreType.DMA,
        pltpu.SemaphoreType.DMA,
        pltpu.SemaphoreType.DMA,
        pltpu.SemaphoreType.DMA,
        pltpu.SemaphoreType.DMA,
        pltpu.SemaphoreType.DMA,
    ],
    compiler_params=_SC_PARAMS,
)(_sc_pass1_body)


def _sc_pass2_body(h2_hbm, src_hbm, dst_hbm, part_out,
                   src_v, dst_v, rows0_v, rows1_v, rows2_v, rows3_v,
                   acc_sh, gsem0, gsem1, gsem2, gsem3,
                   ssem0, ssem1, ssem2, ssem3):
    c = lax.axis_index("c")
    s = lax.axis_index("s")
    wid = c * NS + s
    base = s * RPS

    pltpu.sync_copy(src_hbm.at[wid], src_v)
    pltpu.sync_copy(dst_hbm.at[wid], dst_v)
    _idx_axpy(src_v, 2, 0)

    rows = (rows0_v, rows1_v, rows2_v, rows3_v)
    gsems = (gsem0, gsem1, gsem2, gsem3)
    ssems = (ssem0, ssem1, ssem2, ssem3)
    for h in range(2):
        if h == 1:
            _idx_axpy(src_v, 1, 1)
        for t in range(3):
            _zero_slice(rows0_v, acc_sh, base)
            plsc.subcore_barrier()
            _seg_round(h2_hbm.at[t], src_v, dst_v, rows, gsems, ssems,
                       acc_sh, None)
            plsc.subcore_barrier()

            def wr2(buf_sl, off, ln, t=t, h=h):
                pltpu.sync_copy(buf_sl, part_out.at[
                    t, c, pl.ds(base + off, ln), pl.ds(h * HD, HD)])

            _writeout_slice(rows0_v, acc_sh, base, wr2)


_sc_pass2 = functools.partial(
    pl.kernel,
    out_type=jax.ShapeDtypeStruct((3, NC, N, D), jnp.float32),
    mesh=_MESH,
    scratch_types=[
        pltpu.VMEM((NCH, CH), jnp.int32),
        pltpu.VMEM((NCH, CH), jnp.int32),
        pltpu.VMEM((CH, HD), jnp.float32),
        pltpu.VMEM((CH, HD), jnp.float32),
        pltpu.VMEM((CH, HD), jnp.float32),
        pltpu.VMEM((CH, HD), jnp.float32),
        pltpu.VMEM_SHARED((NACC, HD), jnp.float32),
        pltpu.SemaphoreType.DMA,
        pltpu.SemaphoreType.DMA,
        pltpu.SemaphoreType.DMA,
        pltpu.SemaphoreType.DMA,
        pltpu.SemaphoreType.DMA,
        pltpu.SemaphoreType.DMA,
        pltpu.SemaphoreType.DMA,
        pltpu.SemaphoreType.DMA,
    ],
    compiler_params=_SC_PARAMS,
)(_sc_pass2_body)


BLK = 1000


def _inv_deg(degp):
    deg = degp[0, :, 0:1] + degp[1, :, 0:1]
    return 1.0 / jnp.maximum(deg, 1.0)


def _tc1_body(x_ref, part_ref, degp_ref, w1s_ref, w1n_ref, b1_ref, h_ref):
    x = x_ref[...]
    inv = _inv_deg(degp_ref)
    agg = (part_ref[0] + part_ref[1]) * inv
    for e in range(3):
        h = (jnp.dot(x, w1s_ref[e], preferred_element_type=jnp.float32)
             + jnp.dot(agg, w1n_ref[e], preferred_element_type=jnp.float32)
             + b1_ref[e][None, :])
        h_ref[e] = jnp.maximum(h, 0.0)


_tc1 = pl.pallas_call(
    _tc1_body,
    grid=(N // BLK,),
    in_specs=[
        pl.BlockSpec((BLK, D), lambda i: (i, 0)),
        pl.BlockSpec((NC, BLK, D), lambda i: (0, i, 0)),
        pl.BlockSpec((NC, BLK, DEGW), lambda i: (0, i, 0)),
        pl.BlockSpec((3, D, D), lambda i: (0, 0, 0)),
        pl.BlockSpec((3, D, D), lambda i: (0, 0, 0)),
        pl.BlockSpec((3, D), lambda i: (0, 0)),
    ],
    out_specs=pl.BlockSpec((3, BLK, D), lambda i: (0, i, 0)),
    out_shape=jax.ShapeDtypeStruct((3, N, D), jnp.float32),
)


def _tc2_body(h_ref, part_ref, degp_ref, w2s_ref, w2n_ref, b2_ref, o_ref):
    inv = _inv_deg(degp_ref)
    for e in range(3):
        agg = (part_ref[e, 0] + part_ref[e, 1]) * inv
        o = (jnp.dot(h_ref[e], w2s_ref[e], preferred_element_type=jnp.float32)
             + jnp.dot(agg, w2n_ref[e], preferred_element_type=jnp.float32)
             + b2_ref[e][None, :])
        if e == 1:
            n = jnp.sqrt(jnp.sum(o * o, axis=-1, keepdims=True))
            o = o / jnp.maximum(n, 1e-6)
        elif e == 2:
            n = jnp.maximum(
                jnp.sqrt(jnp.sum(o * o, axis=-1, keepdims=True)), 1e-6)
            o = jnp.tanh(n) * o / n
        o_ref[e] = o


_tc2 = pl.pallas_call(
    _tc2_body,
    grid=(N // BLK,),
    in_specs=[
        pl.BlockSpec((3, BLK, D), lambda i: (0, i, 0)),
        pl.BlockSpec((3, NC, BLK, D), lambda i: (0, 0, i, 0)),
        pl.BlockSpec((NC, BLK, DEGW), lambda i: (0, i, 0)),
        pl.BlockSpec((3, D, D), lambda i: (0, 0, 0)),
        pl.BlockSpec((3, D, D), lambda i: (0, 0, 0)),
        pl.BlockSpec((3, D), lambda i: (0, 0)),
    ],
    out_specs=pl.BlockSpec((3, BLK, D), lambda i: (0, i, 0)),
    out_shape=jax.ShapeDtypeStruct((3, N, D), jnp.float32),
)


def kernel(node_features, edge_index_list, target_node_idx,
           euc_W1s, euc_W1n, euc_b1, euc_W2s, euc_W2n, euc_b2,
           sph_W1s, sph_W1n, sph_b1, sph_W2s, sph_W2n, sph_b2,
           hyp_W1s, hyp_W1n, hyp_b1, hyp_W2s, hyp_W2n, hyp_b2):
    x = node_features
    npad = EPAD - E
    src = jnp.concatenate(
        [edge_index_list[0],
         jnp.zeros((npad,), jnp.int32)]).reshape(NW, NCH, CH)
    dst = jnp.concatenate(
        [edge_index_list[1],
         jnp.full((npad,), N, jnp.int32)]).reshape(NW, NCH, CH)

    part1, degp = _sc_pass1(x.reshape(2 * N, HD), src, dst)

    w1s = jnp.stack([euc_W1s, sph_W1s, hyp_W1s])
    w1n = jnp.stack([euc_W1n, sph_W1n, hyp_W1n])
    b1 = jnp.stack([euc_b1, sph_b1, hyp_b1])
    h = _tc1(x, part1, degp, w1s, w1n, b1)

    part2 = _sc_pass2(h.reshape(3, 2 * N, HD), src, dst)

    w2s = jnp.stack([euc_W2s, sph_W2s, hyp_W2s])
    w2n = jnp.stack([euc_W2n, sph_W2n, hyp_W2n])
    b2 = jnp.stack([euc_b2, sph_b2, hyp_b2])
    return _tc2(h, part2, degp, w2s, w2n, b2)


# speedup vs baseline: 2.9294x; 1.0006x over previous
import functools

import jax
import jax.numpy as jnp
from jax import lax
from jax.experimental import pallas as pl
from jax.experimental.pallas import tpu as pltpu
from jax.experimental.pallas import tpu_sc as plsc

N = 10000
D = 128
HD = 64
E = 320000
NC = 2
NS = 16
NW = NC * NS
CH = 80
NCH = 125
EPW = NCH * CH
EPAD = NW * EPW
RPS = N // NS
DEGW = 16
NACC = N + 8

_MESH = plsc.VectorSubcoreMesh(core_axis_name="c", subcore_axis_name="s",
                               num_cores=NC, num_subcores=NS)
_SC_PARAMS = pltpu.CompilerParams(use_tc_tiling_on_sc=False)


def _zero_rows(ref, nrows, width):
    z = jnp.zeros((16,), jnp.float32)

    def row(i, _):
        for j in range(width // 16):
            ref[i, pl.ds(j * 16, 16)] = z
        return 0

    lax.fori_loop(0, nrows, row, 0)


def _idx_axpy(ref, mul, add):
    def row(i, _):
        for j in range(CH // 16):
            sl = pl.ds(j * 16, 16)
            ref[i, sl] = ref[i, sl] * mul + add
        return 0

    lax.fori_loop(0, NCH, row, 0)


NBUF = 5
_GLEAD = 2
_SLAG = NBUF - _GLEAD


def _seg_round(tab, src_v, dst_v, rows, gsems, ssems, acc_sh, deg_tup):
    def g_start(idx, b):
        pltpu.async_copy(tab.at[src_v.at[idx]], rows[b], gsems[b])

    def g_wait(idx, b):
        pltpu.make_async_copy(tab.at[src_v.at[idx]], rows[b], gsems[b]).wait()

    def s_wait(b):
        pltpu.make_async_copy(
            rows[b], acc_sh.at[dst_v.at[0]], ssems[b]).wait()

    def block(idx, b, wait_s, issue_g):
        g_wait(idx, b)
        pltpu.async_copy(rows[b], acc_sh.at[dst_v.at[idx]], ssems[b],
                         add=True)
        if deg_tup is not None:
            ones_v, deg_sh, dsem = deg_tup
            pltpu.async_copy(ones_v, deg_sh.at[dst_v.at[idx]], dsem,
                             add=True)
            if wait_s:
                pltpu.make_async_copy(
                    ones_v, deg_sh.at[dst_v.at[0]], dsem).wait()
        if wait_s:
            s_wait((b - _SLAG) % NBUF)
        if issue_g:
            g_start(idx + _GLEAD, (b + _GLEAD) % NBUF)

    for i in range(_GLEAD):
        g_start(i, i)
    for i in range(_SLAG):
        block(i, i, False, True)

    def penta(j, _):
        i = NBUF * j + _SLAG
        for k in range(NBUF):
            block(i, (_SLAG + k) % NBUF, True, True)
            i = i + 1
        return 0

    ngrp = (NCH - _SLAG - _GLEAD) // NBUF
    lax.fori_loop(0, ngrp, penta, 0)
    for i in range(_SLAG + NBUF * ngrp, NCH):
        block(i, i % NBUF, True, i + _GLEAD <= NCH - 1)
    for i in range(NCH - _SLAG, NCH):
        s_wait(i % NBUF)
    if deg_tup is not None:
        ones_v, deg_sh, dsem = deg_tup
        for _ in range(_SLAG):
            pltpu.make_async_copy(
                ones_v, deg_sh.at[dst_v.at[0]], dsem).wait()


_WCHUNKS = [(k * CH, CH) for k in range(RPS // CH)] + [
    ((RPS // CH) * CH, RPS - (RPS // CH) * CH)]


def _zero_slice(buf, acc_sh, base):
    _zero_rows(buf, CH, HD)
    for off, ln in _WCHUNKS:
        pltpu.sync_copy(buf.at[pl.ds(0, ln)], acc_sh.at[pl.ds(base + off, ln)])


def _writeout_slice(buf, acc_sh, base, write_fn):
    for off, ln in _WCHUNKS:
        pltpu.sync_copy(acc_sh.at[pl.ds(base + off, ln)], buf.at[pl.ds(0, ln)])
        write_fn(buf.at[pl.ds(0, ln)], off, ln)


def _sc_pass1_body(x2_hbm, src_hbm, dst_hbm, part_out, degp_out,
                   src_v, dst_v, rows0_v, rows1_v, rows2_v, rows3_v, rows4_v,
                   ones_v, deg_v, acc_sh, deg_sh,
                   gsem0, gsem1, gsem2, gsem3, gsem4,
                   ssem0, ssem1, ssem2, ssem3, ssem4, dsem):
    c = lax.axis_index("c")
    s = lax.axis_index("s")
    wid = c * NS + s
    base = s * RPS

    pltpu.sync_copy(src_hbm.at[wid], src_v)
    pltpu.sync_copy(dst_hbm.at[wid], dst_v)
    _idx_axpy(src_v, 2, 0)

    one = jnp.ones((16,), jnp.float32)

    def orow(i, _):
        ones_v[i, pl.ds(0, 16)] = one
        return 0

    lax.fori_loop(0, CH, orow, 0)

    _zero_rows(deg_v, RPS, DEGW)
    pltpu.sync_copy(deg_v, deg_sh.at[pl.ds(base, RPS)])

    rows = (rows0_v, rows1_v, rows2_v, rows3_v, rows4_v)
    gsems = (gsem0, gsem1, gsem2, gsem3, gsem4)
    ssems = (ssem0, ssem1, ssem2, ssem3, ssem4)
    for h in range(2):
        if h == 1:
            _idx_axpy(src_v, 1, 1)
        _zero_slice(rows0_v, acc_sh, base)
        plsc.subcore_barrier()
        _seg_round(x2_hbm, src_v, dst_v, rows, gsems, ssems, acc_sh,
                   (ones_v, deg_sh, dsem) if h == 0 else None)
        plsc.subcore_barrier()

        def wr1(buf_sl, off, ln, h=h):
            pltpu.sync_copy(buf_sl, part_out.at[
                c, pl.ds(base + off, ln), pl.ds(h * HD, HD)])

        _writeout_slice(rows0_v, acc_sh, base, wr1)

    pltpu.sync_copy(deg_sh.at[pl.ds(base, RPS)], deg_v)
    pltpu.sync_copy(deg_v, degp_out.at[c, pl.ds(base, RPS)])


_sc_pass1 = functools.partial(
    pl.kernel,
    out_type=(jax.ShapeDtypeStruct((NC, N, D), jnp.float32),
              jax.ShapeDtypeStruct((NC, N, DEGW), jnp.float32)),
    mesh=_MESH,
    scratch_types=[
        pltpu.VMEM((NCH, CH), jnp.int32),
        pltpu.VMEM((NCH, CH), jnp.int32),
        pltpu.VMEM((CH, HD), jnp.float32),
        pltpu.VMEM((CH, HD), jnp.float32),
        pltpu.VMEM((CH, HD), jnp.float32),
        pltpu.VMEM((CH, HD), jnp.float32),
        pltpu.VMEM((CH, HD), jnp.float32),
        pltpu.VMEM((CH, DEGW), jnp.float32),
        pltpu.VMEM((RPS, DEGW), jnp.float32),
        pltpu.VMEM_SHARED((NACC, HD), jnp.float32),
        pltpu.VMEM_SHARED((NACC, DEGW), jnp.float32),
    ] + [pltpu.SemaphoreType.DMA] * 11,
    compiler_params=_SC_PARAMS,
)(_sc_pass1_body)


def _sc_pass2_body(h2_hbm, src_hbm, dst_hbm, part_out,
                   src_v, dst_v, rows0_v, rows1_v, rows2_v, rows3_v, rows4_v,
                   acc_sh, gsem0, gsem1, gsem2, gsem3, gsem4,
                   ssem0, ssem1, ssem2, ssem3, ssem4):
    c = lax.axis_index("c")
    s = lax.axis_index("s")
    wid = c * NS + s
    base = s * RPS

    pltpu.sync_copy(src_hbm.at[wid], src_v)
    pltpu.sync_copy(dst_hbm.at[wid], dst_v)
    _idx_axpy(src_v, 2, 0)

    rows = (rows0_v, rows1_v, rows2_v, rows3_v, rows4_v)
    gsems = (gsem0, gsem1, gsem2, gsem3, gsem4)
    ssems = (ssem0, ssem1, ssem2, ssem3, ssem4)
    for h in range(2):
        if h == 1:
            _idx_axpy(src_v, 1, 1)
        for t in range(3):
            _zero_slice(rows0_v, acc_sh, base)
            plsc.subcore_barrier()
            _seg_round(h2_hbm.at[t], src_v, dst_v, rows, gsems, ssems,
                       acc_sh, None)
            plsc.subcore_barrier()

            def wr2(buf_sl, off, ln, t=t, h=h):
                pltpu.sync_copy(buf_sl, part_out.at[
                    t, c, pl.ds(base + off, ln), pl.ds(h * HD, HD)])

            _writeout_slice(rows0_v, acc_sh, base, wr2)


_sc_pass2 = functools.partial(
    pl.kernel,
    out_type=jax.ShapeDtypeStruct((3, NC, N, D), jnp.float32),
    mesh=_MESH,
    scratch_types=[
        pltpu.VMEM((NCH, CH), jnp.int32),
        pltpu.VMEM((NCH, CH), jnp.int32),
        pltpu.VMEM((CH, HD), jnp.float32),
        pltpu.VMEM((CH, HD), jnp.float32),
        pltpu.VMEM((CH, HD), jnp.float32),
        pltpu.VMEM((CH, HD), jnp.float32),
        pltpu.VMEM((CH, HD), jnp.float32),
        pltpu.VMEM_SHARED((NACC, HD), jnp.float32),
    ] + [pltpu.SemaphoreType.DMA] * 10,
    compiler_params=_SC_PARAMS,
)(_sc_pass2_body)


BLK = 1000


def _inv_deg(degp):
    deg = degp[0, :, 0:1] + degp[1, :, 0:1]
    return 1.0 / jnp.maximum(deg, 1.0)


def _tc1_body(x_ref, part_ref, degp_ref, w1s_ref, w1n_ref, b1_ref, h_ref):
    x = x_ref[...]
    inv = _inv_deg(degp_ref)
    agg = (part_ref[0] + part_ref[1]) * inv
    for e in range(3):
        h = (jnp.dot(x, w1s_ref[e], preferred_element_type=jnp.float32)
             + jnp.dot(agg, w1n_ref[e], preferred_element_type=jnp.float32)
             + b1_ref[e][None, :])
        h_ref[e] = jnp.maximum(h, 0.0)


_tc1 = pl.pallas_call(
    _tc1_body,
    grid=(N // BLK,),
    in_specs=[
        pl.BlockSpec((BLK, D), lambda i: (i, 0)),
        pl.BlockSpec((NC, BLK, D), lambda i: (0, i, 0)),
        pl.BlockSpec((NC, BLK, DEGW), lambda i: (0, i, 0)),
        pl.BlockSpec((3, D, D), lambda i: (0, 0, 0)),
        pl.BlockSpec((3, D, D), lambda i: (0, 0, 0)),
        pl.BlockSpec((3, D), lambda i: (0, 0)),
    ],
    out_specs=pl.BlockSpec((3, BLK, D), lambda i: (0, i, 0)),
    out_shape=jax.ShapeDtypeStruct((3, N, D), jnp.float32),
)


def _tc2_body(h_ref, part_ref, degp_ref, w2s_ref, w2n_ref, b2_ref, o_ref):
    inv = _inv_deg(degp_ref)
    for e in range(3):
        agg = (part_ref[e, 0] + part_ref[e, 1]) * inv
        o = (jnp.dot(h_ref[e], w2s_ref[e], preferred_element_type=jnp.float32)
             + jnp.dot(agg, w2n_ref[e], preferred_element_type=jnp.float32)
             + b2_ref[e][None, :])
        if e == 1:
            n = jnp.sqrt(jnp.sum(o * o, axis=-1, keepdims=True))
            o = o / jnp.maximum(n, 1e-6)
        elif e == 2:
            n = jnp.maximum(
                jnp.sqrt(jnp.sum(o * o, axis=-1, keepdims=True)), 1e-6)
            o = jnp.tanh(n) * o / n
        o_ref[e] = o


_tc2 = pl.pallas_call(
    _tc2_body,
    grid=(N // BLK,),
    in_specs=[
        pl.BlockSpec((3, BLK, D), lambda i: (0, i, 0)),
        pl.BlockSpec((3, NC, BLK, D), lambda i: (0, 0, i, 0)),
        pl.BlockSpec((NC, BLK, DEGW), lambda i: (0, i, 0)),
        pl.BlockSpec((3, D, D), lambda i: (0, 0, 0)),
        pl.BlockSpec((3, D, D), lambda i: (0, 0, 0)),
        pl.BlockSpec((3, D), lambda i: (0, 0)),
    ],
    out_specs=pl.BlockSpec((3, BLK, D), lambda i: (0, i, 0)),
    out_shape=jax.ShapeDtypeStruct((3, N, D), jnp.float32),
)


def kernel(node_features, edge_index_list, target_node_idx,
           euc_W1s, euc_W1n, euc_b1, euc_W2s, euc_W2n, euc_b2,
           sph_W1s, sph_W1n, sph_b1, sph_W2s, sph_W2n, sph_b2,
           hyp_W1s, hyp_W1n, hyp_b1, hyp_W2s, hyp_W2n, hyp_b2):
    x = node_features
    npad = EPAD - E
    if npad:
        src = jnp.concatenate(
            [edge_index_list[0], jnp.zeros((npad,), jnp.int32)])
        dst = jnp.concatenate(
            [edge_index_list[1], jnp.full((npad,), N, jnp.int32)])
    else:
        src, dst = edge_index_list[0], edge_index_list[1]
    src = src.reshape(NW, NCH, CH)
    dst = dst.reshape(NW, NCH, CH)

    part1, degp = _sc_pass1(x.reshape(2 * N, HD), src, dst)

    w1s = jnp.stack([euc_W1s, sph_W1s, hyp_W1s])
    w1n = jnp.stack([euc_W1n, sph_W1n, hyp_W1n])
    b1 = jnp.stack([euc_b1, sph_b1, hyp_b1])
    h = _tc1(x, part1, degp, w1s, w1n, b1)

    part2 = _sc_pass2(h.reshape(3, 2 * N, HD), src, dst)

    w2s = jnp.stack([euc_W2s, sph_W2s, hyp_W2s])
    w2n = jnp.stack([euc_W2n, sph_W2n, hyp_W2n])
    b2 = jnp.stack([euc_b2, sph_b2, hyp_b2])
    return _tc2(h, part2, degp, w2s, w2n, b2)


# speedup vs baseline: 2.9402x; 1.0037x over previous
import functools

import jax
import jax.numpy as jnp
from jax import lax
from jax.experimental import pallas as pl
from jax.experimental.pallas import tpu as pltpu
from jax.experimental.pallas import tpu_sc as plsc

N = 10000
D = 128
HD = 64
E = 320000
NC = 2
NS = 16
NW = NC * NS
CH = 80
NCH = 125
EPW = NCH * CH
EPAD = NW * EPW
RPS = N // NS
DEGW = 16
NACC = N + 8

_MESH = plsc.VectorSubcoreMesh(core_axis_name="c", subcore_axis_name="s",
                               num_cores=NC, num_subcores=NS)
_SC_PARAMS = pltpu.CompilerParams(use_tc_tiling_on_sc=False)


def _zero_rows(ref, nrows, width):
    z = jnp.zeros((16,), jnp.float32)

    def row(i, _):
        for j in range(width // 16):
            ref[i, pl.ds(j * 16, 16)] = z
        return 0

    lax.fori_loop(0, nrows, row, 0)


def _idx_axpy(ref, mul, add):
    def row(i, _):
        for j in range(CH // 16):
            sl = pl.ds(j * 16, 16)
            ref[i, sl] = ref[i, sl] * mul + add
        return 0

    lax.fori_loop(0, NCH, row, 0)


NBUF = 5
_GLEAD = 2
_SLAG = NBUF - _GLEAD


def _seg_round(tab, src_v, dst_v, rows, gsems, ssems, acc_sh, deg_tup):
    def g_start(idx, b):
        pltpu.async_copy(tab.at[src_v.at[idx]], rows[b], gsems[b])

    def g_wait(idx, b):
        pltpu.make_async_copy(tab.at[src_v.at[idx]], rows[b], gsems[b]).wait()

    def s_wait(b):
        pltpu.make_async_copy(
            rows[b], acc_sh.at[dst_v.at[0]], ssems[b]).wait()

    def block(idx, b, wait_s, issue_g):
        g_wait(idx, b)
        pltpu.async_copy(rows[b], acc_sh.at[dst_v.at[idx]], ssems[b],
                         add=True)
        if deg_tup is not None:
            ones_v, deg_sh, dsem = deg_tup
            pltpu.async_copy(ones_v, deg_sh.at[dst_v.at[idx]], dsem,
                             add=True)
            if wait_s:
                pltpu.make_async_copy(
                    ones_v, deg_sh.at[dst_v.at[0]], dsem).wait()
        if wait_s:
            s_wait((b - _SLAG) % NBUF)
        if issue_g:
            g_start(idx + _GLEAD, (b + _GLEAD) % NBUF)

    for i in range(_GLEAD):
        g_start(i, i)
    for i in range(_SLAG):
        block(i, i, False, True)

    def penta(j, _):
        i = NBUF * j + _SLAG
        for k in range(NBUF):
            block(i, (_SLAG + k) % NBUF, True, True)
            i = i + 1
        return 0

    ngrp = (NCH - _SLAG - _GLEAD) // NBUF
    lax.fori_loop(0, ngrp, penta, 0)
    for i in range(_SLAG + NBUF * ngrp, NCH):
        block(i, i % NBUF, True, i + _GLEAD <= NCH - 1)
    for i in range(NCH - _SLAG, NCH):
        s_wait(i % NBUF)
    if deg_tup is not None:
        ones_v, deg_sh, dsem = deg_tup
        for _ in range(_SLAG):
            pltpu.make_async_copy(
                ones_v, deg_sh.at[dst_v.at[0]], dsem).wait()


_WCHUNKS = [(k * CH, CH) for k in range(RPS // CH)] + [
    ((RPS // CH) * CH, RPS - (RPS // CH) * CH)]


def _zero_slice(buf, acc_sh, base):
    _zero_rows(buf, CH, HD)
    for off, ln in _WCHUNKS:
        pltpu.sync_copy(buf.at[pl.ds(0, ln)], acc_sh.at[pl.ds(base + off, ln)])


def _writeout_slice(buf, acc_sh, base, write_fn):
    del buf
    write_fn(acc_sh.at[pl.ds(base, RPS)], 0, RPS)


def _sc_pass1_body(x2_hbm, src_hbm, dst_hbm, part_out, degp_out,
                   src_v, dst_v, rows0_v, rows1_v, rows2_v, rows3_v, rows4_v,
                   ones_v, deg_v, acc_sh, deg_sh,
                   gsem0, gsem1, gsem2, gsem3, gsem4,
                   ssem0, ssem1, ssem2, ssem3, ssem4, dsem):
    c = lax.axis_index("c")
    s = lax.axis_index("s")
    wid = c * NS + s
    base = s * RPS

    pltpu.sync_copy(src_hbm.at[wid], src_v)
    pltpu.sync_copy(dst_hbm.at[wid], dst_v)
    _idx_axpy(src_v, 2, 0)

    one = jnp.ones((16,), jnp.float32)

    def orow(i, _):
        ones_v[i, pl.ds(0, 16)] = one
        return 0

    lax.fori_loop(0, CH, orow, 0)

    _zero_rows(deg_v, RPS, DEGW)
    pltpu.sync_copy(deg_v, deg_sh.at[pl.ds(base, RPS)])

    rows = (rows0_v, rows1_v, rows2_v, rows3_v, rows4_v)
    gsems = (gsem0, gsem1, gsem2, gsem3, gsem4)
    ssems = (ssem0, ssem1, ssem2, ssem3, ssem4)
    for h in range(2):
        if h == 1:
            _idx_axpy(src_v, 1, 1)
        _zero_slice(rows0_v, acc_sh, base)
        plsc.subcore_barrier()
        _seg_round(x2_hbm, src_v, dst_v, rows, gsems, ssems, acc_sh,
                   (ones_v, deg_sh, dsem) if h == 0 else None)
        plsc.subcore_barrier()

        def wr1(buf_sl, off, ln, h=h):
            pltpu.sync_copy(buf_sl, part_out.at[
                c, pl.ds(base + off, ln), pl.ds(h * HD, HD)])

        _writeout_slice(rows0_v, acc_sh, base, wr1)

    pltpu.sync_copy(deg_sh.at[pl.ds(base, RPS)],
                    degp_out.at[c, pl.ds(base, RPS)])


_sc_pass1 = functools.partial(
    pl.kernel,
    out_type=(jax.ShapeDtypeStruct((NC, N, D), jnp.float32),
              jax.ShapeDtypeStruct((NC, N, DEGW), jnp.float32)),
    mesh=_MESH,
    scratch_types=[
        pltpu.VMEM((NCH, CH), jnp.int32),
        pltpu.VMEM((NCH, CH), jnp.int32),
        pltpu.VMEM((CH, HD), jnp.float32),
        pltpu.VMEM((CH, HD), jnp.float32),
        pltpu.VMEM((CH, HD), jnp.float32),
        pltpu.VMEM((CH, HD), jnp.float32),
        pltpu.VMEM((CH, HD), jnp.float32),
        pltpu.VMEM((CH, DEGW), jnp.float32),
        pltpu.VMEM((RPS, DEGW), jnp.float32),
        pltpu.VMEM_SHARED((NACC, HD), jnp.float32),
        pltpu.VMEM_SHARED((NACC, DEGW), jnp.float32),
    ] + [pltpu.SemaphoreType.DMA] * 11,
    compiler_params=_SC_PARAMS,
)(_sc_pass1_body)


def _sc_pass2_body(h2_hbm, src_hbm, dst_hbm, part_out,
                   src_v, dst_v, rows0_v, rows1_v, rows2_v, rows3_v, rows4_v,
                   acc_sh, gsem0, gsem1, gsem2, gsem3, gsem4,
                   ssem0, ssem1, ssem2, ssem3, ssem4):
    c = lax.axis_index("c")
    s = lax.axis_index("s")
    wid = c * NS + s
    base = s * RPS

    pltpu.sync_copy(src_hbm.at[wid], src_v)
    pltpu.sync_copy(dst_hbm.at[wid], dst_v)
    _idx_axpy(src_v, 2, 0)

    rows = (rows0_v, rows1_v, rows2_v, rows3_v, rows4_v)
    gsems = (gsem0, gsem1, gsem2, gsem3, gsem4)
    ssems = (ssem0, ssem1, ssem2, ssem3, ssem4)
    for h in range(2):
        if h == 1:
            _idx_axpy(src_v, 1, 1)
        for t in range(3):
            _zero_slice(rows0_v, acc_sh, base)
            plsc.subcore_barrier()
            _seg_round(h2_hbm.at[t], src_v, dst_v, rows, gsems, ssems,
                       acc_sh, None)
            plsc.subcore_barrier()

            def wr2(buf_sl, off, ln, t=t, h=h):
                pltpu.sync_copy(buf_sl, part_out.at[
                    t, c, pl.ds(base + off, ln), pl.ds(h * HD, HD)])

            _writeout_slice(rows0_v, acc_sh, base, wr2)


_sc_pass2 = functools.partial(
    pl.kernel,
    out_type=jax.ShapeDtypeStruct((3, NC, N, D), jnp.float32),
    mesh=_MESH,
    scratch_types=[
        pltpu.VMEM((NCH, CH), jnp.int32),
        pltpu.VMEM((NCH, CH), jnp.int32),
        pltpu.VMEM((CH, HD), jnp.float32),
        pltpu.VMEM((CH, HD), jnp.float32),
        pltpu.VMEM((CH, HD), jnp.float32),
        pltpu.VMEM((CH, HD), jnp.float32),
        pltpu.VMEM((CH, HD), jnp.float32),
        pltpu.VMEM_SHARED((NACC, HD), jnp.float32),
    ] + [pltpu.SemaphoreType.DMA] * 10,
    compiler_params=_SC_PARAMS,
)(_sc_pass2_body)


BLK = 1000


def _inv_deg(degp):
    deg = degp[0, :, 0:1] + degp[1, :, 0:1]
    return 1.0 / jnp.maximum(deg, 1.0)


def _tc1_body(x_ref, part_ref, degp_ref, w1s_ref, w1n_ref, b1_ref, h_ref):
    x = x_ref[...]
    inv = _inv_deg(degp_ref)
    agg = (part_ref[0] + part_ref[1]) * inv
    for e in range(3):
        h = (jnp.dot(x, w1s_ref[e], preferred_element_type=jnp.float32)
             + jnp.dot(agg, w1n_ref[e], preferred_element_type=jnp.float32)
             + b1_ref[e][None, :])
        h_ref[e] = jnp.maximum(h, 0.0)


_tc1 = pl.pallas_call(
    _tc1_body,
    grid=(N // BLK,),
    in_specs=[
        pl.BlockSpec((BLK, D), lambda i: (i, 0)),
        pl.BlockSpec((NC, BLK, D), lambda i: (0, i, 0)),
        pl.BlockSpec((NC, BLK, DEGW), lambda i: (0, i, 0)),
        pl.BlockSpec((3, D, D), lambda i: (0, 0, 0)),
        pl.BlockSpec((3, D, D), lambda i: (0, 0, 0)),
        pl.BlockSpec((3, D), lambda i: (0, 0)),
    ],
    out_specs=pl.BlockSpec((3, BLK, D), lambda i: (0, i, 0)),
    out_shape=jax.ShapeDtypeStruct((3, N, D), jnp.float32),
)


def _tc2_body(h_ref, part_ref, degp_ref, w2s_ref, w2n_ref, b2_ref, o_ref):
    inv = _inv_deg(degp_ref)
    for e in range(3):
        agg = (part_ref[e, 0] + part_ref[e, 1]) * inv
        o = (jnp.dot(h_ref[e], w2s_ref[e], preferred_element_type=jnp.float32)
             + jnp.dot(agg, w2n_ref[e], preferred_element_type=jnp.float32)
             + b2_ref[e][None, :])
        if e == 1:
            n = jnp.sqrt(jnp.sum(o * o, axis=-1, keepdims=True))
            o = o / jnp.maximum(n, 1e-6)
        elif e == 2:
            n = jnp.maximum(
                jnp.sqrt(jnp.sum(o * o, axis=-1, keepdims=True)), 1e-6)
            o = jnp.tanh(n) * o / n
        o_ref[e] = o


_tc2 = pl.pallas_call(
    _tc2_body,
    grid=(N // BLK,),
    in_specs=[
        pl.BlockSpec((3, BLK, D), lambda i: (0, i, 0)),
        pl.BlockSpec((3, NC, BLK, D), lambda i: (0, 0, i, 0)),
        pl.BlockSpec((NC, BLK, DEGW), lambda i: (0, i, 0)),
        pl.BlockSpec((3, D, D), lambda i: (0, 0, 0)),
        pl.BlockSpec((3, D, D), lambda i: (0, 0, 0)),
        pl.BlockSpec((3, D), lambda i: (0, 0)),
    ],
    out_specs=pl.BlockSpec((3, BLK, D), lambda i: (0, i, 0)),
    out_shape=jax.ShapeDtypeStruct((3, N, D), jnp.float32),
)


def kernel(node_features, edge_index_list, target_node_idx,
           euc_W1s, euc_W1n, euc_b1, euc_W2s, euc_W2n, euc_b2,
           sph_W1s, sph_W1n, sph_b1, sph_W2s, sph_W2n, sph_b2,
           hyp_W1s, hyp_W1n, hyp_b1, hyp_W2s, hyp_W2n, hyp_b2):
    x = node_features
    npad = EPAD - E
    if npad:
        src = jnp.concatenate(
            [edge_index_list[0], jnp.zeros((npad,), jnp.int32)])
        dst = jnp.concatenate(
            [edge_index_list[1], jnp.full((npad,), N, jnp.int32)])
    else:
        src, dst = edge_index_list[0], edge_index_list[1]
    src = src.reshape(NW, NCH, CH)
    dst = dst.reshape(NW, NCH, CH)

    part1, degp = _sc_pass1(x.reshape(2 * N, HD), src, dst)

    w1s = jnp.stack([euc_W1s, sph_W1s, hyp_W1s])
    w1n = jnp.stack([euc_W1n, sph_W1n, hyp_W1n])
    b1 = jnp.stack([euc_b1, sph_b1, hyp_b1])
    h = _tc1(x, part1, degp, w1s, w1n, b1)

    part2 = _sc_pass2(h.reshape(3, 2 * N, HD), src, dst)

    w2s = jnp.stack([euc_W2s, sph_W2s, hyp_W2s])
    w2n = jnp.stack([euc_W2n, sph_W2n, hyp_W2n])
    b2 = jnp.stack([euc_b2, sph_b2, hyp_b2])
    return _tc2(h, part2, degp, w2s, w2n, b2)
